# Initial kernel scaffold; baseline (speedup 1.0000x reference)
import jax
import jax.numpy as jnp
from jax.experimental import pallas as pl


def kernel(x, edge_index, batch, Wl1, Wr1, b1, Wl2, Wr2, b2, Wl3, Wr3, b3, Wl4, Wr4, b4, Wl5, Wr5, b5, Wo, bo):
    raise NotImplementedError("write your pallas kernel here")


# speedup vs baseline: 3.3709x; 3.3709x over previous
import functools

import jax
import jax.numpy as jnp
from jax import lax
from jax.experimental import pallas as pl
from jax.experimental.pallas import tpu as pltpu
from jax.experimental.pallas import tpu_sc as plsc

N = 10000
E = 320000
D = 128
H = 128
C = 64
G = 128

NC = 2
NS = 16
NW = NC * NS

CH = 128
W_EDGES = 10112
N_CHUNKS = W_EDGES // CH
EP = NW * W_EDGES
NP = 10240
ROWS_PER_SUB = NP // NS
OUT_A = 624
OUT_TAIL = N - NS * OUT_A


@functools.cache
def _make_sc_segment_sum():
  mesh = plsc.VectorSubcoreMesh(core_axis_name="c", subcore_axis_name="s")

  @functools.partial(
      pl.kernel,
      out_type=jax.ShapeDtypeStruct((NC, N, H), jnp.float32),
      mesh=mesh,
      scratch_types=[
          pltpu.VMEM((CH,), jnp.int32),
          pltpu.VMEM((CH,), jnp.int32),
          pltpu.VMEM((CH, H), jnp.float32),
          pltpu.VMEM_SHARED((NP, H), jnp.float32),
          pltpu.SemaphoreType.DMA,
      ],
  )
  def sc_segment_sum(h_hbm, src_hbm, dst_hbm, zeros_hbm, out_hbm,
                     src_v, dst_v, rows_v, acc_sh, sem):
    cid = lax.axis_index("c")
    sid = lax.axis_index("s")

    @pl.loop(0, ROWS_PER_SUB // CH)
    def _(k):
      pltpu.sync_copy(zeros_hbm,
                      acc_sh.at[pl.ds(sid * ROWS_PER_SUB + k * CH, CH)])

    plsc.subcore_barrier()

    wid = sid * NC + cid

    @pl.loop(0, N_CHUNKS)
    def _(i):
      off = wid * W_EDGES + i * CH
      pltpu.sync_copy(src_hbm.at[pl.ds(off, CH)], src_v)
      pltpu.sync_copy(dst_hbm.at[pl.ds(off, CH)], dst_v)
      pltpu.async_copy(h_hbm.at[src_v], rows_v, sem).wait()
      pltpu.sync_copy(rows_v, acc_sh.at[dst_v], add=True)

    plsc.subcore_barrier()

    pltpu.sync_copy(
        acc_sh.at[pl.ds(sid * OUT_A, OUT_A)],
        out_hbm.at[cid].at[pl.ds(sid * OUT_A, OUT_A)],
    )

    @pl.when(sid == 0)
    def _():
      pltpu.sync_copy(
          acc_sh.at[pl.ds(NS * OUT_A, OUT_TAIL)],
          out_hbm.at[cid].at[pl.ds(NS * OUT_A, OUT_TAIL)],
      )

  return sc_segment_sum


@functools.cache
def _make_sc_degree():
  mesh = plsc.VectorSubcoreMesh(core_axis_name="c", subcore_axis_name="s")

  @functools.partial(
      pl.kernel,
      out_type=jax.ShapeDtypeStruct((NC, N, H), jnp.float32),
      mesh=mesh,
      scratch_types=[
          pltpu.VMEM((CH,), jnp.int32),
          pltpu.VMEM((CH, H), jnp.float32),
          pltpu.VMEM_SHARED((NP, H), jnp.float32),
      ],
  )
  def sc_degree(dst_hbm, ones_hbm, zeros_hbm, out_hbm,
                dst_v, ones_v, acc_sh):
    cid = lax.axis_index("c")
    sid = lax.axis_index("s")

    pltpu.sync_copy(ones_hbm, ones_v)

    @pl.loop(0, ROWS_PER_SUB // CH)
    def _(k):
      pltpu.sync_copy(zeros_hbm,
                      acc_sh.at[pl.ds(sid * ROWS_PER_SUB + k * CH, CH)])

    plsc.subcore_barrier()

    wid = sid * NC + cid

    @pl.loop(0, N_CHUNKS)
    def _(i):
      off = wid * W_EDGES + i * CH
      pltpu.sync_copy(dst_hbm.at[pl.ds(off, CH)], dst_v)
      pltpu.sync_copy(ones_v, acc_sh.at[dst_v], add=True)

    plsc.subcore_barrier()

    pltpu.sync_copy(
        acc_sh.at[pl.ds(sid * OUT_A, OUT_A)],
        out_hbm.at[cid].at[pl.ds(sid * OUT_A, OUT_A)],
    )

    @pl.when(sid == 0)
    def _():
      pltpu.sync_copy(
          acc_sh.at[pl.ds(NS * OUT_A, OUT_TAIL)],
          out_hbm.at[cid].at[pl.ds(NS * OUT_A, OUT_TAIL)],
      )

  return sc_degree


_RB = 400
_NB = N // _RB


def _combine_body(m_ref, c_ref, h_ref, wl_ref, wr_ref, b_ref, o_ref):
  cnt = c_ref[0][:, 0:1] + c_ref[1][:, 0:1]
  inv = 1.0 / jnp.maximum(cnt, 1.0)
  agg = (m_ref[0] + m_ref[1]) * inv
  z = (jnp.dot(agg, wl_ref[...], preferred_element_type=jnp.float32)
       + jnp.dot(h_ref[...], wr_ref[...], preferred_element_type=jnp.float32)
       + b_ref[...])
  o_ref[...] = jnp.maximum(z, 0.0)


def _tc_combine(msg, cntp, h, wl, wr, b):
  return pl.pallas_call(
      _combine_body,
      grid=(_NB,),
      in_specs=[
          pl.BlockSpec((NC, _RB, H), lambda i: (0, i, 0)),
          pl.BlockSpec((NC, _RB, H), lambda i: (0, i, 0)),
          pl.BlockSpec((_RB, H), lambda i: (i, 0)),
          pl.BlockSpec((H, H), lambda i: (0, 0)),
          pl.BlockSpec((H, H), lambda i: (0, 0)),
          pl.BlockSpec((1, H), lambda i: (0, 0)),
      ],
      out_specs=pl.BlockSpec((_RB, H), lambda i: (i, 0)),
      out_shape=jax.ShapeDtypeStruct((N, H), jnp.float32),
  )(msg, cntp, h, wl, wr, b.reshape(1, H))


def _pool_body(h_ref, b_ref, wo_ref, bo_ref, o_ref, acc_ref, cacc_ref):
  i = pl.program_id(0)

  @pl.when(i == 0)
  def _():
    acc_ref[...] = jnp.zeros_like(acc_ref)
    cacc_ref[...] = jnp.zeros_like(cacc_ref)

  h = h_ref[...]
  bidx = b_ref[...]
  iota_g = lax.broadcasted_iota(jnp.int32, (_RB, G), 1)
  onehot = (bidx == iota_g).astype(jnp.float32)
  acc_ref[...] += lax.dot_general(
      onehot, h, (((0,), (0,)), ((), ())), preferred_element_type=jnp.float32)
  cacc_ref[...] += lax.dot_general(
      onehot, jnp.ones((_RB, G), jnp.float32), (((0,), (0,)), ((), ())),
      preferred_element_type=jnp.float32)

  @pl.when(i == _NB - 1)
  def _():
    cnt = jnp.maximum(cacc_ref[:, 0:1], 1.0)
    pooled = acc_ref[...] / cnt
    logits = (jnp.dot(pooled, wo_ref[...], preferred_element_type=jnp.float32)
              + bo_ref[...])
    m = jnp.max(logits, axis=1, keepdims=True)
    lse = jnp.log(jnp.sum(jnp.exp(logits - m), axis=1, keepdims=True)) + m
    o_ref[...] = logits - lse


def _tc_pool(h, batch2d, wo, bo):
  return pl.pallas_call(
      _pool_body,
      grid=(_NB,),
      in_specs=[
          pl.BlockSpec((_RB, H), lambda i: (i, 0)),
          pl.BlockSpec((_RB, 1), lambda i: (i, 0)),
          pl.BlockSpec((H, C), lambda i: (0, 0)),
          pl.BlockSpec((1, C), lambda i: (0, 0)),
      ],
      out_specs=pl.BlockSpec((G, C), lambda i: (0, 0)),
      out_shape=jax.ShapeDtypeStruct((G, C), jnp.float32),
      scratch_shapes=[
          pltpu.VMEM((G, H), jnp.float32),
          pltpu.VMEM((G, G), jnp.float32),
      ],
  )(h, batch2d, wo, bo.reshape(1, C))


def kernel(x, edge_index, batch, Wl1, Wr1, b1, Wl2, Wr2, b2, Wl3, Wr3, b3,
           Wl4, Wr4, b4, Wl5, Wr5, b5, Wo, bo):
  src = edge_index[0].astype(jnp.int32)
  dst = edge_index[1].astype(jnp.int32)
  pad = EP - E
  srcp = jnp.concatenate([src, jnp.zeros((pad,), jnp.int32)])
  dstp = jnp.concatenate([dst, jnp.full((pad,), N, jnp.int32)])

  zeros_h = jnp.zeros((CH, H), jnp.float32)
  ones_h = jnp.ones((CH, H), jnp.float32)

  cntp = _make_sc_degree()(dstp, ones_h, zeros_h)

  h = x
  for wl, wr, b in ((Wl1, Wr1, b1), (Wl2, Wr2, b2), (Wl3, Wr3, b3),
                    (Wl4, Wr4, b4), (Wl5, Wr5, b5)):
    msg = _make_sc_segment_sum()(h, srcp, dstp, zeros_h)
    h = _tc_combine(msg, cntp, h, wl, wr, b)

  return _tc_pool(h, batch.astype(jnp.int32).reshape(N, 1), Wo, bo)
